# baseline (speedup 1.0000x reference)
import functools

import jax
import jax.numpy as jnp
from jax import lax
from jax.experimental import pallas as pl
from jax.experimental.pallas import tpu as pltpu
from jax.experimental.pallas import tpu_sc as plsc

LANES = 16
IDX_CHUNK = 128


@functools.lru_cache(maxsize=None)
def _build(batch: int, dim: int):
    info = plsc.get_sparse_core_info()
    num_cores, num_subcores = info.num_cores, info.num_subcores
    num_workers = num_cores * num_subcores
    assert batch % (8 * num_workers) == 0
    b_per_w = batch // num_workers
    assert b_per_w % IDX_CHUNK == 0
    n_chunks = b_per_w // IDX_CHUNK
    n_groups = b_per_w // LANES

    mesh = plsc.VectorSubcoreMesh(core_axis_name="c", subcore_axis_name="s")

    @functools.partial(
        pl.kernel,
        mesh=mesh,
        compiler_params=pltpu.CompilerParams(
            needs_layout_passes=False, use_tc_tiling_on_sc=False),
        out_type=jax.ShapeDtypeStruct((batch,), jnp.float32),
        scratch_types=[
            pltpu.VMEM((b_per_w,), jnp.int32),
            pltpu.VMEM((b_per_w,), jnp.int32),
            pltpu.VMEM((b_per_w, dim), jnp.float32),
            pltpu.VMEM((b_per_w, dim), jnp.float32),
            pltpu.VMEM((b_per_w,), jnp.float32),
            pltpu.VMEM((b_per_w,), jnp.float32),
            pltpu.VMEM((LANES,), jnp.float32),
            pltpu.VMEM((b_per_w,), jnp.float32),
            pltpu.SemaphoreType.DMA,
        ],
    )
    def mf_kernel(uid_hbm, iid_hbm, utab_hbm, itab_hbm, ubias_hbm, ibias_hbm,
                  gmean_hbm, out_hbm, uidx_v, iidx_v, urows_v, irows_v,
                  ub_v, ib_v, gm_v, out_v, sem):
        wid = lax.axis_index("s") * num_cores + lax.axis_index("c")
        base = wid * b_per_w

        pltpu.sync_copy(uid_hbm.at[pl.ds(base, b_per_w)], uidx_v)
        pltpu.sync_copy(iid_hbm.at[pl.ds(base, b_per_w)], iidx_v)
        pltpu.sync_copy(gmean_hbm, gm_v)

        copies = []
        for j in range(n_chunks):
            sl = pl.ds(j * IDX_CHUNK, IDX_CHUNK)
            copies.append(pltpu.async_copy(
                utab_hbm.at[uidx_v.at[sl]], urows_v.at[sl], sem))
            copies.append(pltpu.async_copy(
                itab_hbm.at[iidx_v.at[sl]], irows_v.at[sl], sem))
            copies.append(pltpu.async_copy(
                ubias_hbm.at[uidx_v.at[sl]], ub_v.at[sl], sem))
            copies.append(pltpu.async_copy(
                ibias_hbm.at[iidx_v.at[sl]], ib_v.at[sl], sem))
        for c in copies:
            c.wait()

        gm_vec = gm_v[...]
        lane_iota = lax.iota(jnp.int32, LANES)

        def body(g, carry):
            svec = jnp.zeros((LANES,), jnp.float32)
            for j in range(LANES):
                r = g * LANES + j
                acc = urows_v[r, pl.ds(0, LANES)] * irows_v[r, pl.ds(0, LANES)]
                for c in range(1, dim // LANES):
                    acc = acc + (urows_v[r, pl.ds(c * LANES, LANES)]
                                 * irows_v[r, pl.ds(c * LANES, LANES)])
                svec = jnp.where(lane_iota == j, jnp.sum(acc), svec)
            sl = pl.ds(g * LANES, LANES)
            out_v[sl] = svec + ub_v[sl] + ib_v[sl] + gm_vec
            return carry

        lax.fori_loop(0, n_groups, body, None)

        pltpu.sync_copy(out_v, out_hbm.at[pl.ds(base, b_per_w)])

    return mf_kernel


def kernel(user_ids, item_ids, user_table, item_table, user_bias_table,
           item_bias_table, global_mean):
    batch = user_ids.shape[0]
    dim = user_table.shape[1]
    gm16 = jnp.broadcast_to(jnp.asarray(global_mean, jnp.float32), (LANES,))
    fn = _build(batch, dim)
    return fn(user_ids.astype(jnp.int32), item_ids.astype(jnp.int32),
              user_table, item_table,
              user_bias_table.reshape(-1), item_bias_table.reshape(-1), gm16)
